# Initial kernel scaffold; baseline (speedup 1.0000x reference)
import jax
import jax.numpy as jnp
from jax.experimental import pallas as pl


def kernel(grid_x, grid_edge_index, grid_batch, surf_x, surf_edge_index, surf_batch, W1, b1, W2, b2):
    raise NotImplementedError("write your pallas kernel here")


# speedup vs baseline: 8.6964x; 8.6964x over previous
import jax
import jax.numpy as jnp
from jax import lax
from jax.experimental import pallas as pl
from jax.experimental.pallas import tpu as pltpu
from jax.experimental.pallas import tpu_sc as plsc

_N = 10000
_D = 128
_E = 320000
_G = 64
_NP = 10240
_CH = 128
_NCH = 2560
_EPAD = _NCH * _CH
_NSC = 2
_NSUB = 16
_NW = _NSC * _NSUB
_CPW = _NCH // _NW
_RPT = _NP // _NSUB
_BR = 1280
_NBLK = _NP // _BR

_HIGH = lax.Precision.HIGHEST


def _sc_mesh():
    return plsc.VectorSubcoreMesh(
        core_axis_name="c", subcore_axis_name="s",
        num_cores=_NSC, num_subcores=_NSUB)


def _sc_degree(dst_flat):
    def body(dst_hbm, out_hbm, dst_v, hist):
        c = lax.axis_index("c")
        s = lax.axis_index("s")
        w = c * _NSUB + s
        epw = _CPW * _CH
        pltpu.sync_copy(dst_hbm.at[pl.ds(w * epw, epw)], dst_v)

        def zstep(k, carry):
            hist[pl.ds(k * 16, 16)] = jnp.zeros((16,), jnp.float32)
            return carry

        lax.fori_loop(0, _NP // 16, zstep, 0)
        ones = jnp.ones((16,), jnp.float32)

        def step(i, carry):
            for g in range(_CH // 16):
                idx = dst_v[pl.ds(i * _CH + g * 16, 16)]
                plsc.addupdate_scatter(hist, [idx], ones)
            return carry

        lax.fori_loop(0, _CPW, step, 0)
        pltpu.sync_copy(hist, out_hbm.at[w, 0])

    return pl.kernel(
        body,
        out_type=jax.ShapeDtypeStruct((_NW, 1, _NP), jnp.float32),
        mesh=_sc_mesh(),
        compiler_params=pltpu.CompilerParams(needs_layout_passes=False),
        scratch_types=[
            pltpu.VMEM((_CPW * _CH,), jnp.int32),
            pltpu.VMEM((_NP,), jnp.float32),
        ],
    )(dst_flat)


def _sc_scatter(y, zeros_big, src_r, dst_r):
    def body(y_hbm, zeros_hbm, src_hbm, dst_hbm, out_hbm,
             src_v, dst_v, buf, acc, sem):
        c = lax.axis_index("c")
        s = lax.axis_index("s")
        w = c * _NSUB + s
        r0 = s * _RPT
        pltpu.sync_copy(zeros_hbm.at[pl.ds(r0, _RPT)], acc.at[pl.ds(r0, _RPT)])
        pltpu.sync_copy(src_hbm.at[pl.ds(w * _CPW, _CPW)], src_v)
        pltpu.sync_copy(dst_hbm.at[pl.ds(w * _CPW, _CPW)], dst_v)
        plsc.subcore_barrier()

        def step(i, carry):
            pltpu.async_copy(y_hbm.at[src_v.at[i]], buf, sem).wait()
            pltpu.sync_copy(buf, acc.at[dst_v.at[i]], add=True)
            return carry

        lax.fori_loop(0, _CPW, step, 0)
        plsc.subcore_barrier()
        pltpu.sync_copy(acc.at[pl.ds(r0, _RPT)], out_hbm.at[c, pl.ds(r0, _RPT)])

    return pl.kernel(
        body,
        out_type=jax.ShapeDtypeStruct((_NSC, _NP, _D), jnp.float32),
        mesh=_sc_mesh(),
        scratch_types=[
            pltpu.VMEM((_CPW, _CH), jnp.int32),
            pltpu.VMEM((_CPW, _CH), jnp.int32),
            pltpu.VMEM((_CH, _D), jnp.float32),
            pltpu.VMEM_SHARED((_NP, _D), jnp.float32),
            pltpu.SemaphoreType.DMA,
        ],
    )(y, zeros_big, src_r, dst_r)


def _dinv_block(dp_ref):
    deg = lax.dot_general(dp_ref[:, 0, :], jnp.ones((_NW, 1), jnp.float32),
                          (((0,), (0,)), ((), ())),
                          preferred_element_type=jnp.float32,
                          precision=_HIGH)
    return lax.rsqrt(deg + 1.0)


def _tc_stage1(x, W1, degp):
    def body(x_ref, w_ref, dp_ref, o_ref):
        dinv = _dinv_block(dp_ref)
        xw = jnp.dot(x_ref[...], w_ref[...],
                     preferred_element_type=jnp.float32, precision=_HIGH)
        o_ref[...] = xw * dinv

    return pl.pallas_call(
        body,
        grid=(_NBLK,),
        in_specs=[
            pl.BlockSpec((_BR, _D), lambda i: (i, 0)),
            pl.BlockSpec((_D, _D), lambda i: (0, 0)),
            pl.BlockSpec((_NW, 1, _BR), lambda i: (0, 0, i)),
        ],
        out_specs=pl.BlockSpec((_BR, _D), lambda i: (i, 0)),
        out_shape=jax.ShapeDtypeStruct((_NP, _D), jnp.float32),
    )(x, W1, degp)


def _tc_stage2(y1, parts, W2, b1, degp):
    def body(y_ref, p_ref, w_ref, b_ref, dp_ref, o_ref):
        dinv = _dinv_block(dp_ref)
        tot = p_ref[0] + p_ref[1] + y_ref[...]
        h = jnp.maximum(tot * dinv + b_ref[...], 0.0)
        hw = jnp.dot(h, w_ref[...],
                     preferred_element_type=jnp.float32, precision=_HIGH)
        o_ref[...] = hw * dinv

    return pl.pallas_call(
        body,
        grid=(_NBLK,),
        in_specs=[
            pl.BlockSpec((_BR, _D), lambda i: (i, 0)),
            pl.BlockSpec((_NSC, _BR, _D), lambda i: (0, i, 0)),
            pl.BlockSpec((_D, _D), lambda i: (0, 0)),
            pl.BlockSpec((1, _D), lambda i: (0, 0)),
            pl.BlockSpec((_NW, 1, _BR), lambda i: (0, 0, i)),
        ],
        out_specs=pl.BlockSpec((_BR, _D), lambda i: (i, 0)),
        out_shape=jax.ShapeDtypeStruct((_NP, _D), jnp.float32),
    )(y1, parts, W2, b1, degp)


def _tc_stage3(y2, parts, b2, degp, batchf):
    def body(y_ref, p_ref, b_ref, dp_ref, bt_ref, o_ref, s_sum, s_cnt):
        i = pl.program_id(0)
        dinv = _dinv_block(dp_ref)
        tot = p_ref[0] + p_ref[1] + y_ref[...]
        h = jnp.maximum(tot * dinv + b_ref[...], 0.0)
        seg = bt_ref[0]
        gids = lax.broadcasted_iota(jnp.int32, (_G, _BR), 0)
        mask = jnp.where(seg == gids, 1.0, 0.0)

        @pl.when(i == 0)
        def _():
            s_sum[...] = jnp.zeros_like(s_sum)
            s_cnt[...] = jnp.zeros_like(s_cnt)

        s_sum[...] += jnp.dot(mask, h,
                              preferred_element_type=jnp.float32,
                              precision=_HIGH)
        s_cnt[...] += jnp.sum(mask, axis=1, keepdims=True)

        @pl.when(i == _NBLK - 1)
        def _():
            o_ref[...] = s_sum[...] / jnp.maximum(s_cnt[...], 1.0)

    return pl.pallas_call(
        body,
        grid=(_NBLK,),
        in_specs=[
            pl.BlockSpec((_BR, _D), lambda i: (i, 0)),
            pl.BlockSpec((_NSC, _BR, _D), lambda i: (0, i, 0)),
            pl.BlockSpec((1, _D), lambda i: (0, 0)),
            pl.BlockSpec((_NW, 1, _BR), lambda i: (0, 0, i)),
            pl.BlockSpec((1, 1, _BR), lambda i: (i, 0, 0)),
        ],
        out_specs=pl.BlockSpec((_G, _D), lambda i: (0, 0)),
        out_shape=jax.ShapeDtypeStruct((_G, _D), jnp.float32),
        scratch_shapes=[
            pltpu.VMEM((_G, _D), jnp.float32),
            pltpu.VMEM((_G, 1), jnp.float32),
        ],
    )(y2, parts, b2, degp, batchf)


def kernel(grid_x, grid_edge_index, grid_batch, surf_x, surf_edge_index,
           surf_batch, W1, b1, W2, b2):
    del grid_x, grid_edge_index, grid_batch

    src = surf_edge_index[0]
    dst = surf_edge_index[1]
    src_r = jnp.concatenate(
        [src, jnp.zeros((_EPAD - _E,), jnp.int32)]).reshape(_NCH, _CH)
    dst_r = jnp.concatenate(
        [dst, jnp.full((_EPAD - _E,), _N, jnp.int32)]).reshape(_NCH, _CH)
    x_pad = jnp.pad(surf_x, ((0, _NP - _N), (0, 0)))
    zeros_big = jnp.zeros((_NP, _D), jnp.float32)
    batchf = jnp.pad(surf_batch, (0, _NP - _N),
                     constant_values=_G).reshape(_NBLK, 1, _BR)
    b1r = b1.reshape(1, _D)
    b2r = b2.reshape(1, _D)

    degp = _sc_degree(dst_r.reshape(_EPAD))
    y1 = _tc_stage1(x_pad, W1, degp)
    p1 = _sc_scatter(y1, zeros_big, src_r, dst_r)
    y2 = _tc_stage2(y1, p1, W2, b1r, degp)
    p2 = _sc_scatter(y2, zeros_big, src_r, dst_r)
    return _tc_stage3(y2, p2, b2r, degp, batchf)


# speedup vs baseline: 10.3227x; 1.1870x over previous
import jax
import jax.numpy as jnp
from jax import lax
from jax.experimental import pallas as pl
from jax.experimental.pallas import tpu as pltpu
from jax.experimental.pallas import tpu_sc as plsc

_N = 10000
_D = 128
_E = 320000
_G = 64
_NP = 10240
_CH = 128
_NCH = 2560
_EPAD = _NCH * _CH
_NSC = 2
_NSUB = 16
_NW = _NSC * _NSUB
_CPW = _NCH // _NW
_CPW0 = 120
_CPW1 = 40
_RPT = _NP // _NSUB
_BR = 1280
_NBLK = _NP // _BR

_HIGH = lax.Precision.HIGHEST


def _sc_mesh():
    return plsc.VectorSubcoreMesh(
        core_axis_name="c", subcore_axis_name="s",
        num_cores=_NSC, num_subcores=_NSUB)


def _sc_degree(dst_flat):
    def body(dst_hbm, out_hbm, dst_v, hist):
        c = lax.axis_index("c")
        s = lax.axis_index("s")
        w = c * _NSUB + s
        base = jnp.where(c == 0, s * _CPW0, 16 * _CPW0 + s * _CPW1) * _CH
        ncw = jnp.where(c == 0, _CPW0, _CPW1)
        pltpu.sync_copy(dst_hbm.at[pl.ds(base, _CPW0 * _CH)], dst_v)

        def zstep(k, carry):
            hist[pl.ds(k * 16, 16)] = jnp.zeros((16,), jnp.float32)
            return carry

        lax.fori_loop(0, _NP // 16, zstep, 0)
        ones = jnp.ones((16,), jnp.float32)

        def step(i, carry):
            for g in range(_CH // 16):
                idx = dst_v[pl.ds(i * _CH + g * 16, 16)]
                plsc.addupdate_scatter(hist, [idx], ones)
            return carry

        lax.fori_loop(0, ncw, step, 0)
        pltpu.sync_copy(hist, out_hbm.at[w, 0])

    return pl.kernel(
        body,
        out_type=jax.ShapeDtypeStruct((_NW, 1, _NP), jnp.float32),
        mesh=_sc_mesh(),
        compiler_params=pltpu.CompilerParams(needs_layout_passes=False),
        scratch_types=[
            pltpu.VMEM((_CPW0 * _CH,), jnp.int32),
            pltpu.VMEM((_NP,), jnp.float32),
        ],
    )(dst_flat)


def _sc_scatter(y, zeros_big, src_r, dst_r):
    def body(y_hbm, zeros_hbm, src_hbm, dst_hbm, out_hbm,
             src_v, dst_v, buf, acc, sem):
        c = lax.axis_index("c")
        s = lax.axis_index("s")
        w = c * _NSUB + s
        r0 = s * _RPT
        base = jnp.where(c == 0, s * _CPW0, 16 * _CPW0 + s * _CPW1)
        ncw = jnp.where(c == 0, _CPW0, _CPW1)
        pltpu.sync_copy(zeros_hbm.at[pl.ds(r0, _RPT)], acc.at[pl.ds(r0, _RPT)])
        pltpu.sync_copy(src_hbm.at[pl.ds(base, _CPW0)], src_v)
        pltpu.sync_copy(dst_hbm.at[pl.ds(base, _CPW0)], dst_v)
        plsc.subcore_barrier()

        def step(i, carry):
            pltpu.async_copy(y_hbm.at[src_v.at[i]], buf, sem).wait()
            pltpu.sync_copy(buf, acc.at[dst_v.at[i]], add=True)
            return carry

        lax.fori_loop(0, ncw, step, 0)
        plsc.subcore_barrier()
        pltpu.sync_copy(acc.at[pl.ds(r0, _RPT)], out_hbm.at[c, pl.ds(r0, _RPT)])

    return pl.kernel(
        body,
        out_type=jax.ShapeDtypeStruct((_NSC, _NP, _D), jnp.float32),
        mesh=_sc_mesh(),
        scratch_types=[
            pltpu.VMEM((_CPW0, _CH), jnp.int32),
            pltpu.VMEM((_CPW0, _CH), jnp.int32),
            pltpu.VMEM((_CH, _D), jnp.float32),
            pltpu.VMEM_SHARED((_NP, _D), jnp.float32),
            pltpu.SemaphoreType.DMA,
        ],
    )(y, zeros_big, src_r, dst_r)


def _dinv_block(dp_ref):
    deg = lax.dot_general(dp_ref[:, 0, :], jnp.ones((_NW, 1), jnp.float32),
                          (((0,), (0,)), ((), ())),
                          preferred_element_type=jnp.float32,
                          precision=_HIGH)
    return lax.rsqrt(deg + 1.0)


def _tc_stage1(x, W1, degp):
    def body(x_ref, w_ref, dp_ref, o_ref):
        dinv = _dinv_block(dp_ref)
        xw = jnp.dot(x_ref[...], w_ref[...],
                     preferred_element_type=jnp.float32, precision=_HIGH)
        o_ref[...] = xw * dinv

    return pl.pallas_call(
        body,
        grid=(_NBLK,),
        in_specs=[
            pl.BlockSpec((_BR, _D), lambda i: (i, 0)),
            pl.BlockSpec((_D, _D), lambda i: (0, 0)),
            pl.BlockSpec((_NW, 1, _BR), lambda i: (0, 0, i)),
        ],
        out_specs=pl.BlockSpec((_BR, _D), lambda i: (i, 0)),
        out_shape=jax.ShapeDtypeStruct((_NP, _D), jnp.float32),
    )(x, W1, degp)


def _tc_stage2(y1, parts, W2, b1, degp):
    def body(y_ref, p_ref, w_ref, b_ref, dp_ref, o_ref):
        dinv = _dinv_block(dp_ref)
        tot = p_ref[0] + p_ref[1] + y_ref[...]
        h = jnp.maximum(tot * dinv + b_ref[...], 0.0)
        hw = jnp.dot(h, w_ref[...],
                     preferred_element_type=jnp.float32, precision=_HIGH)
        o_ref[...] = hw * dinv

    return pl.pallas_call(
        body,
        grid=(_NBLK,),
        in_specs=[
            pl.BlockSpec((_BR, _D), lambda i: (i, 0)),
            pl.BlockSpec((_NSC, _BR, _D), lambda i: (0, i, 0)),
            pl.BlockSpec((_D, _D), lambda i: (0, 0)),
            pl.BlockSpec((1, _D), lambda i: (0, 0)),
            pl.BlockSpec((_NW, 1, _BR), lambda i: (0, 0, i)),
        ],
        out_specs=pl.BlockSpec((_BR, _D), lambda i: (i, 0)),
        out_shape=jax.ShapeDtypeStruct((_NP, _D), jnp.float32),
    )(y1, parts, W2, b1, degp)


def _tc_stage3(y2, parts, b2, degp, batchf):
    def body(y_ref, p_ref, b_ref, dp_ref, bt_ref, o_ref, s_sum, s_cnt):
        i = pl.program_id(0)
        dinv = _dinv_block(dp_ref)
        tot = p_ref[0] + p_ref[1] + y_ref[...]
        h = jnp.maximum(tot * dinv + b_ref[...], 0.0)
        seg = bt_ref[0]
        gids = lax.broadcasted_iota(jnp.int32, (_G, _BR), 0)
        mask = jnp.where(seg == gids, 1.0, 0.0)

        @pl.when(i == 0)
        def _():
            s_sum[...] = jnp.zeros_like(s_sum)
            s_cnt[...] = jnp.zeros_like(s_cnt)

        s_sum[...] += jnp.dot(mask, h,
                              preferred_element_type=jnp.float32,
                              precision=_HIGH)
        s_cnt[...] += jnp.sum(mask, axis=1, keepdims=True)

        @pl.when(i == _NBLK - 1)
        def _():
            o_ref[...] = s_sum[...] / jnp.maximum(s_cnt[...], 1.0)

    return pl.pallas_call(
        body,
        grid=(_NBLK,),
        in_specs=[
            pl.BlockSpec((_BR, _D), lambda i: (i, 0)),
            pl.BlockSpec((_NSC, _BR, _D), lambda i: (0, i, 0)),
            pl.BlockSpec((1, _D), lambda i: (0, 0)),
            pl.BlockSpec((_NW, 1, _BR), lambda i: (0, 0, i)),
            pl.BlockSpec((1, 1, _BR), lambda i: (i, 0, 0)),
        ],
        out_specs=pl.BlockSpec((_G, _D), lambda i: (0, 0)),
        out_shape=jax.ShapeDtypeStruct((_G, _D), jnp.float32),
        scratch_shapes=[
            pltpu.VMEM((_G, _D), jnp.float32),
            pltpu.VMEM((_G, 1), jnp.float32),
        ],
    )(y2, parts, b2, degp, batchf)


def kernel(grid_x, grid_edge_index, grid_batch, surf_x, surf_edge_index,
           surf_batch, W1, b1, W2, b2):
    del grid_x, grid_edge_index, grid_batch

    src = surf_edge_index[0]
    dst = surf_edge_index[1]
    pad_n = _EPAD - _E + _CPW0 * _CH
    src_r = jnp.concatenate(
        [src, jnp.zeros((pad_n,), jnp.int32)]).reshape(_NCH + _CPW0, _CH)
    dst_r = jnp.concatenate(
        [dst, jnp.full((pad_n,), _N, jnp.int32)]).reshape(_NCH + _CPW0, _CH)
    x_pad = jnp.pad(surf_x, ((0, _NP - _N), (0, 0)))
    zeros_big = jnp.zeros((_NP, _D), jnp.float32)
    batchf = jnp.pad(surf_batch, (0, _NP - _N),
                     constant_values=_G).reshape(_NBLK, 1, _BR)
    b1r = b1.reshape(1, _D)
    b2r = b2.reshape(1, _D)

    degp = _sc_degree(dst_r.reshape(_EPAD + _CPW0 * _CH))
    y1 = _tc_stage1(x_pad, W1, degp)
    p1 = _sc_scatter(y1, zeros_big, src_r, dst_r)
    y2 = _tc_stage2(y1, p1, W2, b1r, degp)
    p2 = _sc_scatter(y2, zeros_big, src_r, dst_r)
    return _tc_stage3(y2, p2, b2r, degp, batchf)


# speedup vs baseline: 10.6232x; 1.0291x over previous
import jax
import jax.numpy as jnp
from jax import lax
from jax.experimental import pallas as pl
from jax.experimental.pallas import tpu as pltpu
from jax.experimental.pallas import tpu_sc as plsc

_N = 10000
_D = 128
_E = 320000
_G = 64
_NP = 10240
_CH = 128
_NCH = 2560
_EPAD = _NCH * _CH
_NSC = 2
_NSUB = 16
_NW = _NSC * _NSUB
_CPW = _NCH // _NW
_CPW0 = 120
_CPW1 = 40
_RPT = _NP // _NSUB
_BR = 1280
_NBLK = _NP // _BR

_HIGH = lax.Precision.HIGHEST


def _sc_mesh():
    return plsc.VectorSubcoreMesh(
        core_axis_name="c", subcore_axis_name="s",
        num_cores=_NSC, num_subcores=_NSUB)


def _sc_degree(dst_flat):
    def body(dst_hbm, out_hbm, dst_v, hist):
        c = lax.axis_index("c")
        s = lax.axis_index("s")
        w = c * _NSUB + s
        base = jnp.where(c == 0, s * _CPW0, 16 * _CPW0 + s * _CPW1) * _CH
        ncw = jnp.where(c == 0, _CPW0, _CPW1)
        pltpu.sync_copy(dst_hbm.at[pl.ds(base, _CPW0 * _CH)], dst_v)

        def zstep(k, carry):
            hist[pl.ds(k * 16, 16)] = jnp.zeros((16,), jnp.float32)
            return carry

        lax.fori_loop(0, _NP // 16, zstep, 0)
        ones = jnp.ones((16,), jnp.float32)

        def step(i, carry):
            for g in range(_CH // 16):
                idx = dst_v[pl.ds(i * _CH + g * 16, 16)]
                plsc.addupdate_scatter(hist, [idx], ones)
            return carry

        lax.fori_loop(0, ncw, step, 0)
        pltpu.sync_copy(hist, out_hbm.at[w, 0])

    return pl.kernel(
        body,
        out_type=jax.ShapeDtypeStruct((_NW, 1, _NP), jnp.float32),
        mesh=_sc_mesh(),
        compiler_params=pltpu.CompilerParams(needs_layout_passes=False),
        scratch_types=[
            pltpu.VMEM((_CPW0 * _CH,), jnp.int32),
            pltpu.VMEM((_NP,), jnp.float32),
        ],
    )(dst_flat)


def _sc_scatter(y, zeros_big, src_r, dst_r):
    def body(y_hbm, zeros_hbm, src_hbm, dst_hbm, out_hbm,
             src_v, dst_v, buf, acc, sem):
        c = lax.axis_index("c")
        s = lax.axis_index("s")
        w = c * _NSUB + s
        r0 = s * _RPT
        base = jnp.where(c == 0, s * _CPW0, 16 * _CPW0 + s * _CPW1)
        ncw = jnp.where(c == 0, _CPW0, _CPW1)
        def zstep(j, carry):
            pltpu.sync_copy(zeros_hbm, acc.at[pl.ds(r0 + j * _CH, _CH)])
            return carry

        lax.fori_loop(0, _RPT // _CH, zstep, 0)
        pltpu.sync_copy(src_hbm.at[pl.ds(base, _CPW0)], src_v)
        pltpu.sync_copy(dst_hbm.at[pl.ds(base, _CPW0)], dst_v)
        plsc.subcore_barrier()

        def step(i, carry):
            pltpu.async_copy(y_hbm.at[src_v.at[i]], buf, sem).wait()
            pltpu.sync_copy(buf, acc.at[dst_v.at[i]], add=True)
            return carry

        lax.fori_loop(0, ncw, step, 0)
        plsc.subcore_barrier()
        pltpu.sync_copy(acc.at[pl.ds(r0, _RPT)], out_hbm.at[c, pl.ds(r0, _RPT)])

    return pl.kernel(
        body,
        out_type=jax.ShapeDtypeStruct((_NSC, _NP, _D), jnp.float32),
        mesh=_sc_mesh(),
        scratch_types=[
            pltpu.VMEM((_CPW0, _CH), jnp.int32),
            pltpu.VMEM((_CPW0, _CH), jnp.int32),
            pltpu.VMEM((_CH, _D), jnp.float32),
            pltpu.VMEM_SHARED((_NP, _D), jnp.float32),
            pltpu.SemaphoreType.DMA,
        ],
    )(y, zeros_big, src_r, dst_r)


def _dinv_block(dp_ref):
    deg = lax.dot_general(dp_ref[:, 0, :], jnp.ones((_NW, 1), jnp.float32),
                          (((0,), (0,)), ((), ())),
                          preferred_element_type=jnp.float32,
                          precision=_HIGH)
    return lax.rsqrt(deg + 1.0)


def _tc_stage1(x, W1, degp):
    def body(x_ref, w_ref, dp_ref, o_ref):
        dinv = _dinv_block(dp_ref)
        xw = jnp.dot(x_ref[...], w_ref[...],
                     preferred_element_type=jnp.float32, precision=_HIGH)
        o_ref[...] = xw * dinv

    return pl.pallas_call(
        body,
        grid=(_NBLK,),
        in_specs=[
            pl.BlockSpec((_BR, _D), lambda i: (i, 0)),
            pl.BlockSpec((_D, _D), lambda i: (0, 0)),
            pl.BlockSpec((_NW, 1, _BR), lambda i: (0, 0, i)),
        ],
        out_specs=pl.BlockSpec((_BR, _D), lambda i: (i, 0)),
        out_shape=jax.ShapeDtypeStruct((_NP, _D), jnp.float32),
    )(x, W1, degp)


def _tc_stage2(y1, parts, W2, b1, degp):
    def body(y_ref, p_ref, w_ref, b_ref, dp_ref, o_ref):
        dinv = _dinv_block(dp_ref)
        tot = p_ref[0] + p_ref[1] + y_ref[...]
        h = jnp.maximum(tot * dinv + b_ref[...], 0.0)
        hw = jnp.dot(h, w_ref[...],
                     preferred_element_type=jnp.float32, precision=_HIGH)
        o_ref[...] = hw * dinv

    return pl.pallas_call(
        body,
        grid=(_NBLK,),
        in_specs=[
            pl.BlockSpec((_BR, _D), lambda i: (i, 0)),
            pl.BlockSpec((_NSC, _BR, _D), lambda i: (0, i, 0)),
            pl.BlockSpec((_D, _D), lambda i: (0, 0)),
            pl.BlockSpec((1, _D), lambda i: (0, 0)),
            pl.BlockSpec((_NW, 1, _BR), lambda i: (0, 0, i)),
        ],
        out_specs=pl.BlockSpec((_BR, _D), lambda i: (i, 0)),
        out_shape=jax.ShapeDtypeStruct((_NP, _D), jnp.float32),
    )(y1, parts, W2, b1, degp)


def _tc_stage3(y2, parts, b2, degp, batchf):
    def body(y_ref, p_ref, b_ref, dp_ref, bt_ref, o_ref, s_sum, s_cnt):
        i = pl.program_id(0)
        dinv = _dinv_block(dp_ref)
        tot = p_ref[0] + p_ref[1] + y_ref[...]
        h = jnp.maximum(tot * dinv + b_ref[...], 0.0)
        seg = bt_ref[0]
        gids = lax.broadcasted_iota(jnp.int32, (_G, _BR), 0)
        mask = jnp.where(seg == gids, 1.0, 0.0)

        @pl.when(i == 0)
        def _():
            s_sum[...] = jnp.zeros_like(s_sum)
            s_cnt[...] = jnp.zeros_like(s_cnt)

        s_sum[...] += jnp.dot(mask, h,
                              preferred_element_type=jnp.float32,
                              precision=_HIGH)
        s_cnt[...] += jnp.sum(mask, axis=1, keepdims=True)

        @pl.when(i == _NBLK - 1)
        def _():
            o_ref[...] = s_sum[...] / jnp.maximum(s_cnt[...], 1.0)

    return pl.pallas_call(
        body,
        grid=(_NBLK,),
        in_specs=[
            pl.BlockSpec((_BR, _D), lambda i: (i, 0)),
            pl.BlockSpec((_NSC, _BR, _D), lambda i: (0, i, 0)),
            pl.BlockSpec((1, _D), lambda i: (0, 0)),
            pl.BlockSpec((_NW, 1, _BR), lambda i: (0, 0, i)),
            pl.BlockSpec((1, 1, _BR), lambda i: (i, 0, 0)),
        ],
        out_specs=pl.BlockSpec((_G, _D), lambda i: (0, 0)),
        out_shape=jax.ShapeDtypeStruct((_G, _D), jnp.float32),
        scratch_shapes=[
            pltpu.VMEM((_G, _D), jnp.float32),
            pltpu.VMEM((_G, 1), jnp.float32),
        ],
    )(y2, parts, b2, degp, batchf)


def kernel(grid_x, grid_edge_index, grid_batch, surf_x, surf_edge_index,
           surf_batch, W1, b1, W2, b2):
    del grid_x, grid_edge_index, grid_batch

    src = surf_edge_index[0]
    dst = surf_edge_index[1]
    pad_n = _EPAD - _E + _CPW0 * _CH
    src_r = jnp.concatenate(
        [src, jnp.zeros((pad_n,), jnp.int32)]).reshape(_NCH + _CPW0, _CH)
    dst_r = jnp.concatenate(
        [dst, jnp.full((pad_n,), _N, jnp.int32)]).reshape(_NCH + _CPW0, _CH)
    x_pad = jnp.pad(surf_x, ((0, _NP - _N), (0, 0)))
    zeros_sm = jnp.zeros((_CH, _D), jnp.float32)
    batchf = jnp.pad(surf_batch, (0, _NP - _N),
                     constant_values=_G).reshape(_NBLK, 1, _BR)
    b1r = b1.reshape(1, _D)
    b2r = b2.reshape(1, _D)

    degp = _sc_degree(dst_r.reshape(_EPAD + _CPW0 * _CH))
    y1 = _tc_stage1(x_pad, W1, degp)
    p1 = _sc_scatter(y1, zeros_sm, src_r, dst_r)
    y2 = _tc_stage2(y1, p1, W2, b1r, degp)
    p2 = _sc_scatter(y2, zeros_sm, src_r, dst_r)
    return _tc_stage3(y2, p2, b2r, degp, batchf)
